# baseline (speedup 1.0000x reference)
import jax
import jax.numpy as jnp
from jax.experimental import pallas as pl

TOKENS = 32768
HIDDEN = 768
E = 64
TOPK = 8

BT = 512


def _router_block(x_ref, w_ref, b_ref, logits_ref, wts_ref, idx_ref, mask_ref):
    logits = jnp.dot(x_ref[...], w_ref[...], preferred_element_type=jnp.float32)
    logits = logits + b_ref[...]
    lt = logits.T
    logits_ref[...] = lt

    sub_i = jax.lax.broadcasted_iota(jnp.int32, (E, BT), 0)
    keys = lt

    ms = []
    ids = []
    for k in range(TOPK):
        m = jnp.max(keys, axis=0, keepdims=True)
        imin = jnp.min(
            jnp.where(keys == m, sub_i, E), axis=0, keepdims=True
        )
        sel1 = sub_i == imin
        mask_ref[k, :, :] = sel1.astype(jnp.int32)
        ms.append(m)
        ids.append(imin)
        if k + 1 < TOPK:
            keys = jnp.where(sel1, -jnp.inf, keys)

    mtop = jnp.concatenate(ms, axis=0)
    idx_ref[...] = jnp.concatenate(ids, axis=0)

    ex = jnp.exp(mtop - ms[0])
    wts_ref[...] = ex / jnp.sum(ex, axis=0, keepdims=True)


@jax.jit
def kernel(hidden_states, W, b):
    grid = (TOKENS // BT,)
    out_shapes = (
        jax.ShapeDtypeStruct((E, TOKENS), jnp.float32),
        jax.ShapeDtypeStruct((TOPK, TOKENS), jnp.float32),
        jax.ShapeDtypeStruct((TOPK, TOKENS), jnp.int32),
        jax.ShapeDtypeStruct((TOPK, E, TOKENS), jnp.int32),
    )
    b2 = b.reshape(1, E)
    logits_t, wts_t, idx_t, mask_t = pl.pallas_call(
        _router_block,
        grid=grid,
        in_specs=[
            pl.BlockSpec((BT, HIDDEN), lambda i: (i, 0)),
            pl.BlockSpec((HIDDEN, E), lambda i: (0, 0)),
            pl.BlockSpec((1, E), lambda i: (0, 0)),
        ],
        out_specs=(
            pl.BlockSpec((E, BT), lambda i: (0, i)),
            pl.BlockSpec((TOPK, BT), lambda i: (0, i)),
            pl.BlockSpec((TOPK, BT), lambda i: (0, i)),
            pl.BlockSpec((TOPK, E, BT), lambda i: (0, 0, i)),
        ),
        out_shape=out_shapes,
    )(hidden_states, W, b2)
    return (
        logits_t.T,
        wts_t.T,
        idx_t.T,
        jnp.transpose(mask_t, (2, 0, 1)),
    )


# speedup vs baseline: 1.3385x; 1.3385x over previous
import jax
import jax.numpy as jnp
from jax.experimental import pallas as pl

TOKENS = 32768
HIDDEN = 768
E = 64
TOPK = 8

BT = 1024


def _router_block(x_ref, w_ref, b_ref, logits_ref, wts_ref, idx_ref, mask_ref):
    logits = jnp.dot(x_ref[...], w_ref[...], preferred_element_type=jnp.float32)
    logits = logits + b_ref[...]
    lt = logits.T
    logits_ref[...] = lt

    sub_i = jax.lax.broadcasted_iota(jnp.int32, (E, BT), 0)
    keys = lt

    ms = []
    ids = []
    for k in range(TOPK):
        m = jnp.max(keys, axis=0, keepdims=True)
        imin = jnp.min(
            jnp.where(keys == m, sub_i, E), axis=0, keepdims=True
        )
        sel1 = sub_i == imin
        mask_ref[k, :, :] = sel1.astype(jnp.int32)
        ms.append(m)
        ids.append(imin)
        if k + 1 < TOPK:
            keys = jnp.where(sel1, -jnp.inf, keys)

    mtop = jnp.concatenate(ms, axis=0)
    idx_ref[...] = jnp.concatenate(ids, axis=0)

    ex = jnp.exp(mtop - ms[0])
    wts_ref[...] = ex / jnp.sum(ex, axis=0, keepdims=True)


@jax.jit
def kernel(hidden_states, W, b):
    grid = (TOKENS // BT,)
    out_shapes = (
        jax.ShapeDtypeStruct((E, TOKENS), jnp.float32),
        jax.ShapeDtypeStruct((TOPK, TOKENS), jnp.float32),
        jax.ShapeDtypeStruct((TOPK, TOKENS), jnp.int32),
        jax.ShapeDtypeStruct((TOPK, E, TOKENS), jnp.int32),
    )
    b2 = b.reshape(1, E)
    logits_t, wts_t, idx_t, mask_t = pl.pallas_call(
        _router_block,
        grid=grid,
        in_specs=[
            pl.BlockSpec((BT, HIDDEN), lambda i: (i, 0)),
            pl.BlockSpec((HIDDEN, E), lambda i: (0, 0)),
            pl.BlockSpec((1, E), lambda i: (0, 0)),
        ],
        out_specs=(
            pl.BlockSpec((E, BT), lambda i: (0, i)),
            pl.BlockSpec((TOPK, BT), lambda i: (0, i)),
            pl.BlockSpec((TOPK, BT), lambda i: (0, i)),
            pl.BlockSpec((TOPK, E, BT), lambda i: (0, 0, i)),
        ),
        out_shape=out_shapes,
    )(hidden_states, W, b2)
    return (
        logits_t.T,
        wts_t.T,
        idx_t.T,
        jnp.transpose(mask_t, (2, 0, 1)),
    )


# speedup vs baseline: 1.4722x; 1.0999x over previous
import jax
import jax.numpy as jnp
from jax.experimental import pallas as pl

TOKENS = 32768
HIDDEN = 768
E = 64
TOPK = 8

BT = 2048


def _router_block(x_ref, w_ref, b_ref, logits_ref, wts_ref, idx_ref, mask_ref):
    logits = jnp.dot(x_ref[...], w_ref[...], preferred_element_type=jnp.float32)
    logits = logits + b_ref[...]
    lt = logits.T
    logits_ref[...] = lt

    sub_i = jax.lax.broadcasted_iota(jnp.int32, (E, BT), 0)
    keys = lt

    ms = []
    ids = []
    for k in range(TOPK):
        m = jnp.max(keys, axis=0, keepdims=True)
        imin = jnp.min(
            jnp.where(keys == m, sub_i, E), axis=0, keepdims=True
        )
        sel1 = sub_i == imin
        mask_ref[k, :, :] = sel1.astype(jnp.int32)
        ms.append(m)
        ids.append(imin)
        if k + 1 < TOPK:
            keys = jnp.where(sel1, -jnp.inf, keys)

    mtop = jnp.concatenate(ms, axis=0)
    idx_ref[...] = jnp.concatenate(ids, axis=0)

    ex = jnp.exp(mtop - ms[0])
    wts_ref[...] = ex / jnp.sum(ex, axis=0, keepdims=True)


@jax.jit
def kernel(hidden_states, W, b):
    grid = (TOKENS // BT,)
    out_shapes = (
        jax.ShapeDtypeStruct((E, TOKENS), jnp.float32),
        jax.ShapeDtypeStruct((TOPK, TOKENS), jnp.float32),
        jax.ShapeDtypeStruct((TOPK, TOKENS), jnp.int32),
        jax.ShapeDtypeStruct((TOPK, E, TOKENS), jnp.int32),
    )
    b2 = b.reshape(1, E)
    logits_t, wts_t, idx_t, mask_t = pl.pallas_call(
        _router_block,
        grid=grid,
        in_specs=[
            pl.BlockSpec((BT, HIDDEN), lambda i: (i, 0)),
            pl.BlockSpec((HIDDEN, E), lambda i: (0, 0)),
            pl.BlockSpec((1, E), lambda i: (0, 0)),
        ],
        out_specs=(
            pl.BlockSpec((E, BT), lambda i: (0, i)),
            pl.BlockSpec((TOPK, BT), lambda i: (0, i)),
            pl.BlockSpec((TOPK, BT), lambda i: (0, i)),
            pl.BlockSpec((TOPK, E, BT), lambda i: (0, 0, i)),
        ),
        out_shape=out_shapes,
    )(hidden_states, W, b2)
    return (
        logits_t.T,
        wts_t.T,
        idx_t.T,
        jnp.transpose(mask_t, (2, 0, 1)),
    )


# speedup vs baseline: 1.5263x; 1.0368x over previous
import jax
import jax.numpy as jnp
from jax.experimental import pallas as pl

TOKENS = 32768
HIDDEN = 768
E = 64
TOPK = 8

BT = 4096


def _router_block(x_ref, w_ref, b_ref, logits_ref, wts_ref, idx_ref, mask_ref):
    logits = jnp.dot(x_ref[...], w_ref[...], preferred_element_type=jnp.float32)
    logits = logits + b_ref[...]
    lt = logits.T
    logits_ref[...] = lt

    sub_i = jax.lax.broadcasted_iota(jnp.int32, (E, BT), 0)
    keys = lt

    ms = []
    ids = []
    for k in range(TOPK):
        m = jnp.max(keys, axis=0, keepdims=True)
        imin = jnp.min(
            jnp.where(keys == m, sub_i, E), axis=0, keepdims=True
        )
        sel1 = sub_i == imin
        mask_ref[k, :, :] = sel1.astype(jnp.int32)
        ms.append(m)
        ids.append(imin)
        if k + 1 < TOPK:
            keys = jnp.where(sel1, -jnp.inf, keys)

    mtop = jnp.concatenate(ms, axis=0)
    idx_ref[...] = jnp.concatenate(ids, axis=0)

    ex = jnp.exp(mtop - ms[0])
    wts_ref[...] = ex / jnp.sum(ex, axis=0, keepdims=True)


@jax.jit
def kernel(hidden_states, W, b):
    grid = (TOKENS // BT,)
    out_shapes = (
        jax.ShapeDtypeStruct((E, TOKENS), jnp.float32),
        jax.ShapeDtypeStruct((TOPK, TOKENS), jnp.float32),
        jax.ShapeDtypeStruct((TOPK, TOKENS), jnp.int32),
        jax.ShapeDtypeStruct((TOPK, E, TOKENS), jnp.int32),
    )
    b2 = b.reshape(1, E)
    logits_t, wts_t, idx_t, mask_t = pl.pallas_call(
        _router_block,
        grid=grid,
        in_specs=[
            pl.BlockSpec((BT, HIDDEN), lambda i: (i, 0)),
            pl.BlockSpec((HIDDEN, E), lambda i: (0, 0)),
            pl.BlockSpec((1, E), lambda i: (0, 0)),
        ],
        out_specs=(
            pl.BlockSpec((E, BT), lambda i: (0, i)),
            pl.BlockSpec((TOPK, BT), lambda i: (0, i)),
            pl.BlockSpec((TOPK, BT), lambda i: (0, i)),
            pl.BlockSpec((TOPK, E, BT), lambda i: (0, 0, i)),
        ),
        out_shape=out_shapes,
    )(hidden_states, W, b2)
    return (
        logits_t.T,
        wts_t.T,
        idx_t.T,
        jnp.transpose(mask_t, (2, 0, 1)),
    )
